# baseline (speedup 1.0000x reference)
import functools

import jax
import jax.numpy as jnp
from jax import lax
from jax.experimental import pallas as pl
from jax.experimental.pallas import tpu as pltpu
from jax.experimental.pallas import tpu_sc as plsc

_L = 16
_NW = 32
_BW = 128


@functools.lru_cache(maxsize=None)
def _build(B, S, D):
    NG = _BW // _L
    NP = S // 2 - 1
    NQ = (NP + 1) // 2
    mesh = plsc.VectorSubcoreMesh(core_axis_name="c", subcore_axis_name="s")

    @functools.partial(
        pl.kernel,
        out_type=jax.ShapeDtypeStruct((S, D, B), jnp.float32),
        mesh=mesh,
        scratch_types=[
            pltpu.VMEM((S, _BW), jnp.int32),
            pltpu.VMEM((4, D, _BW), jnp.float32),
            pltpu.VMEM((4, D, _BW), jnp.float32),
            pltpu.VMEM((2, D, _BW), jnp.float32),
            pltpu.VMEM((2, D, _BW), jnp.float32),
            pltpu.SemaphoreType.DMA,
            pltpu.SemaphoreType.DMA,
            pltpu.SemaphoreType.DMA,
            pltpu.SemaphoreType.DMA,
            pltpu.SemaphoreType.DMA,
        ],
        compiler_params=pltpu.CompilerParams(needs_layout_passes=False),
    )
    def lor_kernel(xp, obs, out, obs_v, xq0, xq1, cb0, cb1,
                   sem_obs, sem_x0, sem_x1, sem_c0, sem_c1):
        wid = lax.axis_index("s") * 2 + lax.axis_index("c")
        b0 = wid * _BW
        xqs = (xq0, xq1)
        cbs = (cb0, cb1)
        sem_xs = (sem_x0, sem_x1)
        sem_cs = (sem_c0, sem_c1)

        def quad_dma(j, jp):
            s0 = jnp.minimum(4 * j + 1, S - 4)
            return pltpu.make_async_copy(
                xp.at[pl.ds(s0, 4), :, pl.ds(b0, _BW)], xqs[jp], sem_xs[jp]
            )

        def x0_dma(slot):
            return pltpu.make_async_copy(
                xp.at[pl.ds(0, 1), :, pl.ds(b0, _BW)],
                cbs[1].at[pl.ds(slot, 1)], sem_cs[1]
            )

        def out_dma(q, t0):
            return pltpu.make_async_copy(
                cbs[q], out.at[pl.ds(t0, 2), :, pl.ds(b0, _BW)], sem_cs[q]
            )

        def do_pair(p, q, xbuf, sl0, first=False):
            t0 = 2 * p + 2
            if not first:
                out_dma(q, t0 - 4).wait()
            cprev = cbs[1 - q]
            cdst = cbs[q]
            m0 = [obs_v[t0 - 1, pl.ds(k * _L, _L)] != 0 for k in range(NG)]
            m1 = [obs_v[t0, pl.ds(k * _L, _L)] != 0 for k in range(NG)]

            @plsc.parallel_loop(0, D, step=1, unroll=8)
            def _upd0(d):
                for k in range(NG):
                    sl = pl.ds(k * _L, _L)
                    cdst[0, d, sl] = jnp.where(
                        m0[k], xbuf[sl0, d, sl], cprev[1, d, sl]
                    )

            @plsc.parallel_loop(0, D, step=1, unroll=8)
            def _upd1(d):
                for k in range(NG):
                    sl = pl.ds(k * _L, _L)
                    cdst[1, d, sl] = jnp.where(
                        m1[k], xbuf[sl0 + 1, d, sl], cdst[0, d, sl]
                    )

            out_dma(q, t0).start()

        def do_quad(j, jp, first=False):
            quad_dma(j + 1, 1 - jp).start()
            quad_dma(j, jp).wait()
            do_pair(2 * j, 0, xqs[jp], 0, first=first)
            do_pair(2 * j + 1, 1, xqs[jp], 2)

        obs_cp = pltpu.make_async_copy(
            obs.at[:, pl.ds(b0, _BW)], obs_v, sem_obs
        )
        obs_cp.start()
        x0_dma(0).start()
        x0_dma(1).start()
        quad_dma(0, 0).start()
        x0_dma(0).wait()
        x0_dma(1).wait()
        out_dma(1, 0).start()
        obs_cp.wait()

        do_quad(0, 0, first=True)

        def body(i, acc):
            do_quad(2 * i + 1, 1)
            do_quad(2 * i + 2, 0)
            return acc

        lax.fori_loop(0, (NQ - 2) // 2, body, 0)

        jl = NQ - 1
        quad_dma(jl, jl % 2).wait()
        do_pair(2 * jl, 0, xqs[jl % 2], 1)

        out_dma(1, S - 4).wait()
        out_dma(0, S - 2).wait()

    return lor_kernel


def kernel(x, observed):
    B, S, D = x.shape
    xp = jnp.transpose(x, (1, 2, 0))
    obsT = jnp.transpose(observed.astype(jnp.int32), (1, 0))
    outp = _build(B, S, D)(xp, obsT)
    return jnp.transpose(outp, (2, 0, 1))
